# baseline (speedup 1.0000x reference)
import functools

import jax
import jax.numpy as jnp
from jax import lax
from jax.experimental import pallas as pl
from jax.experimental.pallas import tpu as pltpu
from jax.experimental.pallas import tpu_sc as plsc

VOCAB = 1000000
EMBED = 128
NUM_DENSE = 13
N_SPARSE = 26
B = 4096
NF = N_SPARSE + 1

NW = 32
ROWS = B * N_SPARSE // 128
CPW = ROWS // NW


def _sc_gather(idx1, table):
    mesh = plsc.VectorSubcoreMesh(core_axis_name="c", subcore_axis_name="s")
    nrows = idx1.shape[0] // 128
    cpw = nrows // NW
    ipw = cpw * 128

    @functools.partial(
        pl.kernel,
        out_type=jax.ShapeDtypeStruct((nrows, 128, EMBED), jnp.float32),
        mesh=mesh,
        scratch_types=[
            pltpu.VMEM((ipw,), jnp.int32),
            pltpu.VMEM((128, EMBED), jnp.float32),
            pltpu.VMEM((128, EMBED), jnp.float32),
            pltpu.SemaphoreType.DMA,
            pltpu.SemaphoreType.DMA,
        ],
    )
    def gk(idx_hbm, tab_hbm, out_hbm, idx_v, buf0, buf1, sem0, sem1):
        wid = lax.axis_index("s") * 2 + lax.axis_index("c")
        base = wid * cpw
        pltpu.sync_copy(idx_hbm.at[pl.ds(pl.multiple_of(wid * ipw, 128), ipw)],
                        idx_v)

        def body(g, carry):
            c0 = 2 * g
            o0 = pl.multiple_of(c0 * 128, 128)
            o1 = pl.multiple_of(c0 * 128 + 128, 128)
            d0 = pltpu.async_copy(
                tab_hbm.at[idx_v.at[pl.ds(o0, 128)]], buf0, sem0)
            d1 = pltpu.async_copy(
                tab_hbm.at[idx_v.at[pl.ds(o1, 128)]], buf1, sem1)
            d0.wait()
            pltpu.sync_copy(buf0, out_hbm.at[base + c0])
            d1.wait()
            pltpu.sync_copy(buf1, out_hbm.at[base + c0 + 1])
            return carry

        lax.fori_loop(0, cpw // 2, body, 0)
        if cpw % 2:
            c = cpw - 1
            o = pl.multiple_of(c * 128, 128)
            pltpu.async_copy(
                tab_hbm.at[idx_v.at[pl.ds(o, 128)]], buf0, sem0).wait()
            pltpu.sync_copy(buf0, out_hbm.at[base + c])

    return gk(idx1, table)


_BB = 512


def _dT(w, x):
    return lax.dot_general(w, x, (((0,), (0,)), ((), ())),
                           preferred_element_type=jnp.float32)


def _tc_body(denseT_ref, emb_ref, bW0_ref, bb0_ref, bW1_ref, bb1_ref,
             bW2_ref, bb2_ref, tW0_ref, tb0_ref, tW1_ref,
             tb1_ref, tW2_ref, tb2_ref, tW3_ref, tb3_ref, tW4_ref,
             tb4_ref, out_ref):
    bf = jnp.bfloat16
    h = jnp.maximum(_dT(bW0_ref[...], denseT_ref[...]) + bb0_ref[...], 0.0)
    h = jnp.maximum(_dT(bW1_ref[...], h) + bb1_ref[...], 0.0)
    botT = jnp.maximum(_dT(bW2_ref[...], h) + bb2_ref[...], 0.0)

    embT = emb_ref[...].astype(bf).T
    fT = jnp.concatenate([botT.astype(bf), embT], axis=0)
    f3 = fT.reshape(NF, EMBED, _BB)

    xrows = []
    for i in range(NF):
        prod = f3[i:] * f3[i][None]
        xrows.append(jnp.sum(prod, axis=1))
    xT = jnp.concatenate(xrows, axis=0)

    y = (_dT(tW0_ref[EMBED:], xT.astype(jnp.float32)) + _dT(tW0_ref[:EMBED], botT)
         + tb0_ref[...])
    t = jnp.maximum(y, 0.0)
    t = jnp.maximum(_dT(tW1_ref[...], t) + tb1_ref[...], 0.0)
    t = jnp.maximum(_dT(tW2_ref[...], t) + tb2_ref[...], 0.0)
    t = jnp.maximum(_dT(tW3_ref[...], t) + tb3_ref[...], 0.0)
    out_ref[...] = _dT(tW4_ref[...], t) + tb4_ref[...]


def _full_spec(arr):
    nd = arr.ndim
    return pl.BlockSpec(arr.shape, lambda i, _n=nd: (0,) * _n)


def _tc_forward(denseT, emb2, weights):
    nb = denseT.shape[1]
    grid = (nb // _BB,)
    in_specs = [
        pl.BlockSpec((denseT.shape[0], _BB), lambda i: (0, i)),
        pl.BlockSpec((_BB, emb2.shape[1]), lambda i: (i, 0)),
    ] + [_full_spec(w) for w in weights]
    return pl.pallas_call(
        _tc_body,
        grid=grid,
        in_specs=in_specs,
        out_specs=pl.BlockSpec((1, _BB), lambda i: (0, i)),
        out_shape=jax.ShapeDtypeStruct((1, nb), jnp.float32),
    )(denseT, emb2, *weights)


def kernel(x, embedding_table, bW0, bb0, bW1, bb1, bW2, bb2,
           tW0, tb0, tW1, tb1, tW2, tb2, tW3, tb3, tW4, tb4):
    dense = x[:, :NUM_DENSE]
    cat = x[:, NUM_DENSE:]
    idx1 = (jnp.asarray(cat, jnp.int32) % VOCAB).reshape(-1)

    half = B // 2 * N_SPARSE
    embs = [
        _sc_gather(idx1[:half], embedding_table).reshape(B // 2, -1),
        _sc_gather(idx1[half:], embedding_table).reshape(B // 2, -1),
    ]

    weights = (bW0, bb0.reshape(-1, 1), bW1, bb1.reshape(-1, 1),
               bW2, bb2.reshape(-1, 1), tW0, tb0.reshape(-1, 1),
               tW1, tb1.reshape(-1, 1), tW2, tb2.reshape(-1, 1),
               tW3, tb3.reshape(-1, 1), tW4, tb4.reshape(-1, 1))

    dT = dense.T
    outs = [_tc_forward(dT[:, :B // 2], embs[0], weights),
            _tc_forward(dT[:, B // 2:], embs[1], weights)]
    return jnp.concatenate(outs, axis=1).T


# speedup vs baseline: 1.0399x; 1.0399x over previous
import functools

import jax
import jax.numpy as jnp
from jax import lax
from jax.experimental import pallas as pl
from jax.experimental.pallas import tpu as pltpu
from jax.experimental.pallas import tpu_sc as plsc

VOCAB = 1000000
EMBED = 128
NUM_DENSE = 13
N_SPARSE = 26
B = 4096
NF = N_SPARSE + 1

NW = 32
ROWS = B * N_SPARSE // 128
CPW = ROWS // NW


def _sc_gather(idx1, table):
    mesh = plsc.VectorSubcoreMesh(core_axis_name="c", subcore_axis_name="s")
    ipw = CPW * 128

    @functools.partial(
        pl.kernel,
        out_type=jax.ShapeDtypeStruct((ROWS, 128, EMBED), jnp.float32),
        mesh=mesh,
        scratch_types=[
            pltpu.VMEM((ipw,), jnp.int32),
            pltpu.VMEM((128, EMBED), jnp.float32),
            pltpu.VMEM((128, EMBED), jnp.float32),
            pltpu.SemaphoreType.DMA,
            pltpu.SemaphoreType.DMA,
        ],
    )
    def gk(idx_hbm, tab_hbm, out_hbm, idx_v, buf0, buf1, sem0, sem1):
        wid = lax.axis_index("s") * 2 + lax.axis_index("c")
        base = wid * CPW
        pltpu.sync_copy(idx_hbm.at[pl.ds(pl.multiple_of(wid * ipw, 128), ipw)],
                        idx_v)

        def body(g, carry):
            c0 = 2 * g
            o0 = pl.multiple_of(c0 * 128, 128)
            o1 = pl.multiple_of(c0 * 128 + 128, 128)
            d0 = pltpu.async_copy(
                tab_hbm.at[idx_v.at[pl.ds(o0, 128)]], buf0, sem0)
            d1 = pltpu.async_copy(
                tab_hbm.at[idx_v.at[pl.ds(o1, 128)]], buf1, sem1)
            d0.wait()
            pltpu.sync_copy(buf0, out_hbm.at[base + c0])
            d1.wait()
            pltpu.sync_copy(buf1, out_hbm.at[base + c0 + 1])
            return carry

        lax.fori_loop(0, CPW // 2, body, 0)

    return gk(idx1, table)


_BB = 512


def _dT(w, x):
    return lax.dot_general(w, x, (((0,), (0,)), ((), ())),
                           preferred_element_type=jnp.float32)


def _tc_body(denseT_ref, emb_ref, bW0_ref, bb0_ref, bW1_ref, bb1_ref,
             bW2_ref, bb2_ref, tW0_ref, tb0_ref, tW1_ref,
             tb1_ref, tW2_ref, tb2_ref, tW3_ref, tb3_ref, tW4_ref,
             tb4_ref, out_ref):
    h = jnp.maximum(_dT(bW0_ref[...], denseT_ref[...]) + bb0_ref[...], 0.0)
    h = jnp.maximum(_dT(bW1_ref[...], h) + bb1_ref[...], 0.0)
    botT = jnp.maximum(_dT(bW2_ref[...], h) + bb2_ref[...], 0.0)

    embT = emb_ref[...].T
    fT = jnp.concatenate([botT, embT], axis=0)
    f3 = fT.reshape(NF, EMBED, _BB)

    xrows = []
    for i in range(NF):
        prod = f3[i:] * f3[i][None]
        xrows.append(jnp.sum(prod, axis=1))
    xT = jnp.concatenate(xrows, axis=0)

    y = (_dT(tW0_ref[EMBED:], xT) + _dT(tW0_ref[:EMBED], botT)
         + tb0_ref[...])
    t = jnp.maximum(y, 0.0)
    t = jnp.maximum(_dT(tW1_ref[...], t) + tb1_ref[...], 0.0)
    t = jnp.maximum(_dT(tW2_ref[...], t) + tb2_ref[...], 0.0)
    t = jnp.maximum(_dT(tW3_ref[...], t) + tb3_ref[...], 0.0)
    out_ref[...] = _dT(tW4_ref[...], t) + tb4_ref[...]


def _full_spec(arr):
    nd = arr.ndim
    return pl.BlockSpec(arr.shape, lambda i, _n=nd: (0,) * _n)


def _tc_forward(denseT, emb2, weights):
    grid = (B // _BB,)
    in_specs = [
        pl.BlockSpec((denseT.shape[0], _BB), lambda i: (0, i)),
        pl.BlockSpec((_BB, emb2.shape[1]), lambda i: (i, 0)),
    ] + [_full_spec(w) for w in weights]
    return pl.pallas_call(
        _tc_body,
        grid=grid,
        in_specs=in_specs,
        out_specs=pl.BlockSpec((1, _BB), lambda i: (0, i)),
        out_shape=jax.ShapeDtypeStruct((1, B), jnp.float32),
    )(denseT, emb2, *weights)


def kernel(x, embedding_table, bW0, bb0, bW1, bb1, bW2, bb2,
           tW0, tb0, tW1, tb1, tW2, tb2, tW3, tb3, tW4, tb4):
    dense = x[:, :NUM_DENSE]
    cat = x[:, NUM_DENSE:]
    idx1 = (jnp.asarray(cat, jnp.int32) % VOCAB).reshape(-1)

    emb = _sc_gather(idx1, embedding_table).reshape(B, N_SPARSE * EMBED)

    weights = (bW0, bb0.reshape(-1, 1), bW1, bb1.reshape(-1, 1),
               bW2, bb2.reshape(-1, 1), tW0, tb0.reshape(-1, 1),
               tW1, tb1.reshape(-1, 1), tW2, tb2.reshape(-1, 1),
               tW3, tb3.reshape(-1, 1), tW4, tb4.reshape(-1, 1))

    out = _tc_forward(dense.T, emb, weights)
    return out.T


# speedup vs baseline: 1.1916x; 1.1459x over previous
import functools

import jax
import jax.numpy as jnp
from jax import lax
from jax.experimental import pallas as pl
from jax.experimental.pallas import tpu as pltpu
from jax.experimental.pallas import tpu_sc as plsc

VOCAB = 1000000
EMBED = 128
NUM_DENSE = 13
N_SPARSE = 26
B = 4096
NF = N_SPARSE + 1

NW = 32
ROWS = B * N_SPARSE // 128
CPW = ROWS // NW


def _sc_gather(idx1, table):
    mesh = plsc.VectorSubcoreMesh(core_axis_name="c", subcore_axis_name="s")
    ipw = CPW * 128

    @functools.partial(
        pl.kernel,
        out_type=jax.ShapeDtypeStruct((ROWS, 128, EMBED), jnp.float32),
        mesh=mesh,
        scratch_types=[
            pltpu.VMEM((ipw,), jnp.int32),
            pltpu.VMEM((128, EMBED), jnp.float32),
            pltpu.VMEM((128, EMBED), jnp.float32),
            pltpu.SemaphoreType.DMA,
            pltpu.SemaphoreType.DMA,
        ],
    )
    def gk(idx_hbm, tab_hbm, out_hbm, idx_v, buf0, buf1, sem0, sem1):
        wid = lax.axis_index("s") * 2 + lax.axis_index("c")
        base = wid * CPW
        pltpu.sync_copy(idx_hbm.at[pl.ds(pl.multiple_of(wid * ipw, 128), ipw)],
                        idx_v)

        def body(g, carry):
            c0 = 2 * g
            o0 = pl.multiple_of(c0 * 128, 128)
            o1 = pl.multiple_of(c0 * 128 + 128, 128)
            d0 = pltpu.async_copy(
                tab_hbm.at[idx_v.at[pl.ds(o0, 128)]], buf0, sem0)
            d1 = pltpu.async_copy(
                tab_hbm.at[idx_v.at[pl.ds(o1, 128)]], buf1, sem1)
            d0.wait()
            pltpu.sync_copy(buf0, out_hbm.at[base + c0])
            d1.wait()
            pltpu.sync_copy(buf1, out_hbm.at[base + c0 + 1])
            return carry

        lax.fori_loop(0, CPW // 2, body, 0)

    return gk(idx1, table)


_BB = 512


def _dT(w, x):
    return lax.dot_general(w, x, (((0,), (0,)), ((), ())),
                           preferred_element_type=jnp.float32)


def _tc_body(denseT_ref, emb_ref, bW0_ref, bb0_ref, bW1_ref, bb1_ref,
             bW2_ref, bb2_ref, tW0_ref, tb0_ref, tW1_ref,
             tb1_ref, tW2_ref, tb2_ref, tW3_ref, tb3_ref, tW4_ref,
             tb4_ref, out_ref):
    h = jnp.maximum(_dT(bW0_ref[...], denseT_ref[...]) + bb0_ref[...], 0.0)
    h = jnp.maximum(_dT(bW1_ref[...], h) + bb1_ref[...], 0.0)
    botT = jnp.maximum(_dT(bW2_ref[...], h) + bb2_ref[...], 0.0)

    bf = jnp.bfloat16
    embT = emb_ref[...].astype(bf).T
    fT = jnp.concatenate([botT.astype(bf), embT], axis=0)
    f3 = fT.reshape(NF, EMBED, _BB)

    xrows = []
    for i in range(NF):
        prod = f3[i:] * f3[i][None]
        s = jnp.sum(prod, axis=1, dtype=bf)
        xrows.append(s.astype(jnp.float32))
    xT = jnp.concatenate(xrows, axis=0)

    y = (_dT(tW0_ref[EMBED:], xT) + _dT(tW0_ref[:EMBED], botT)
         + tb0_ref[...])
    t = jnp.maximum(y, 0.0)
    t = jnp.maximum(_dT(tW1_ref[...], t) + tb1_ref[...], 0.0)
    t = jnp.maximum(_dT(tW2_ref[...], t) + tb2_ref[...], 0.0)
    t = jnp.maximum(_dT(tW3_ref[...], t) + tb3_ref[...], 0.0)
    out_ref[...] = _dT(tW4_ref[...], t) + tb4_ref[...]


def _full_spec(arr):
    nd = arr.ndim
    return pl.BlockSpec(arr.shape, lambda i, _n=nd: (0,) * _n)


def _tc_forward(denseT, emb2, weights):
    grid = (B // _BB,)
    in_specs = [
        pl.BlockSpec((denseT.shape[0], _BB), lambda i: (0, i)),
        pl.BlockSpec((_BB, emb2.shape[1]), lambda i: (i, 0)),
    ] + [_full_spec(w) for w in weights]
    return pl.pallas_call(
        _tc_body,
        grid=grid,
        in_specs=in_specs,
        out_specs=pl.BlockSpec((1, _BB), lambda i: (0, i)),
        out_shape=jax.ShapeDtypeStruct((1, B), jnp.float32),
    )(denseT, emb2, *weights)


def kernel(x, embedding_table, bW0, bb0, bW1, bb1, bW2, bb2,
           tW0, tb0, tW1, tb1, tW2, tb2, tW3, tb3, tW4, tb4):
    dense = x[:, :NUM_DENSE]
    cat = x[:, NUM_DENSE:]
    idx1 = (jnp.asarray(cat, jnp.int32) % VOCAB).reshape(-1)

    emb = _sc_gather(idx1, embedding_table).reshape(B, N_SPARSE * EMBED)

    weights = (bW0, bb0.reshape(-1, 1), bW1, bb1.reshape(-1, 1),
               bW2, bb2.reshape(-1, 1), tW0, tb0.reshape(-1, 1),
               tW1, tb1.reshape(-1, 1), tW2, tb2.reshape(-1, 1),
               tW3, tb3.reshape(-1, 1), tW4, tb4.reshape(-1, 1))

    out = _tc_forward(dense.T, emb, weights)
    return out.T
